# baseline (speedup 1.0000x reference)
import functools

import jax
import jax.numpy as jnp
from jax.experimental import pallas as pl
from jax.experimental.pallas import tpu as pltpu
from jax.experimental.pallas import tpu_sc as plsc

E = 64
D = 768
DFF = 2048
T = 2048
BT = 64
NB = T // BT + E
TP = NB * BT

_NC, _NS = 2, 16
_NW = _NC * _NS


def _routing_body(x_ref, rw_ref, eid_ref, wt_ref):
    x = x_ref[...]
    rw = rw_ref[...]
    logits = jax.lax.dot_general(
        x, rw, (((1,), (1,)), ((), ())),
        preferred_element_type=jnp.float32,
        precision=jax.lax.Precision.DEFAULT,
    )
    lmax = jnp.max(logits, axis=1, keepdims=True)
    sumexp = jnp.sum(jnp.exp(logits - lmax), axis=1, keepdims=True)
    iota = jax.lax.broadcasted_iota(jnp.int32, (T, E), 1)
    eid = jnp.min(jnp.where(logits == lmax, iota, E), axis=1, keepdims=True)
    eid_ref[...] = eid
    wt_ref[...] = 1.0 / sumexp


def _make_sc_row_gather(n_out, chunk):
    per_w = n_out // _NW
    nchunks = per_w // chunk
    mesh = plsc.VectorSubcoreMesh(
        core_axis_name="c",
        subcore_axis_name="s",
        num_cores=_NC,
        num_subcores=_NS,
    )

    @functools.partial(
        pl.kernel,
        mesh=mesh,
        out_type=jax.ShapeDtypeStruct((n_out, D), jnp.float32),
        scratch_types=[
            pltpu.VMEM((chunk,), jnp.int32),
            pltpu.VMEM((chunk, D), jnp.float32),
            pltpu.SemaphoreType.DMA,
        ],
    )
    def k(src_hbm, idx_hbm, out_hbm, idx_v, rows_v, sem):
        wid = jax.lax.axis_index("s") * _NC + jax.lax.axis_index("c")
        base = wid * per_w
        for c in range(nchunks):
            off = base + c * chunk
            pltpu.sync_copy(idx_hbm.at[pl.ds(off, chunk)], idx_v)
            pltpu.async_copy(src_hbm.at[idx_v], rows_v, sem).wait()
            pltpu.sync_copy(rows_v, out_hbm.at[pl.ds(off, chunk)])

    return k


_combine_gather = _make_sc_row_gather(T, BT)


def _y_copy(ybuf, y_hbm, sem, slot, row):
    return pltpu.make_async_copy(
        ybuf.at[slot], y_hbm.at[pl.ds(row * BT, BT), :], sem.at[slot]
    )


def _moe_body(
    nblk_ref,
    sblk_ref,
    tok_ref,
    wblk_ref,
    x_ref,
    g_hbm,
    u_hbm,
    d_hbm,
    y_hbm,
    gbuf,
    ubuf,
    dbuf,
    ybuf,
    wsem,
    ysem,
):
    def w_copies(ei, sl):
        cs = []
        for p in range(4):
            ff = pl.ds(p * (DFF // 4), DFF // 4)
            dd = pl.ds(p * (D // 4), D // 4)
            cs.append(
                pltpu.make_async_copy(
                    g_hbm.at[ei, ff, :], gbuf.at[sl, ff, :], wsem.at[sl]
                )
            )
            cs.append(
                pltpu.make_async_copy(
                    u_hbm.at[ei, ff, :], ubuf.at[sl, ff, :], wsem.at[sl]
                )
            )
            cs.append(
                pltpu.make_async_copy(
                    d_hbm.at[ei, dd, :], dbuf.at[sl, dd, :], wsem.at[sl]
                )
            )
        return cs

    for c in w_copies(0, 0):
        c.start()

    def expert_step(e, carry):
        slot = jax.lax.rem(e, 2)

        @pl.when(e + 1 < E)
        def _():
            for c in w_copies(e + 1, jax.lax.rem(e + 1, 2)):
                c.start()

        for c in w_copies(e, slot):
            c.wait()

        gw = gbuf[slot]
        uw = ubuf[slot]
        dw = dbuf[slot]
        s0 = sblk_ref[e]
        n = nblk_ref[e]

        def sub_block(k, kc):
            row = s0 + k
            yslot = jax.lax.rem(k, 2)
            idx = tok_ref[pl.ds(row, 1), :][0]
            w = wblk_ref[pl.ds(row, 1), :][0]
            iota_bt = jax.lax.broadcasted_iota(jnp.int32, (BT, T), 1)
            gat = (iota_bt == idx[:, None]).astype(jnp.float32)
            xb = jax.lax.dot_general(
                gat, x_ref[...], (((1,), (0,)), ((), ())),
                preferred_element_type=jnp.float32,
                precision=jax.lax.Precision.DEFAULT,
            )
            hg = jax.lax.dot_general(
                xb, gw, (((1,), (1,)), ((), ())),
                preferred_element_type=jnp.float32,
                precision=jax.lax.Precision.DEFAULT,
            )
            hu = jax.lax.dot_general(
                xb, uw, (((1,), (1,)), ((), ())),
                preferred_element_type=jnp.float32,
                precision=jax.lax.Precision.DEFAULT,
            )
            h = hg * jax.nn.sigmoid(hg) * hu
            y = jax.lax.dot_general(
                h, dw, (((1,), (1,)), ((), ())),
                preferred_element_type=jnp.float32,
                precision=jax.lax.Precision.DEFAULT,
            )

            @pl.when(k >= 2)
            def _():
                _y_copy(ybuf, y_hbm, ysem, yslot, row - 2).wait()

            ybuf[pl.ds(yslot, 1), :, :] = (y * w[:, None])[None]
            _y_copy(ybuf, y_hbm, ysem, yslot, row).start()
            return kc

        jax.lax.fori_loop(0, n, sub_block, 0)

        @pl.when(n >= 2)
        def _():
            _y_copy(ybuf, y_hbm, ysem, jax.lax.rem(n - 2, 2), s0 + n - 2).wait()

        @pl.when(n >= 1)
        def _():
            _y_copy(ybuf, y_hbm, ysem, jax.lax.rem(n - 1, 2), s0 + n - 1).wait()

        return carry

    jax.lax.fori_loop(0, E, expert_step, 0)


@jax.jit
def kernel(hidden_states, router_W, gate_W, up_W, down_W):
    b, s, d = hidden_states.shape
    x = hidden_states.reshape(-1, d).astype(jnp.float32)

    eid2, wt2 = pl.pallas_call(
        _routing_body,
        out_shape=(
            jax.ShapeDtypeStruct((T, 1), jnp.int32),
            jax.ShapeDtypeStruct((T, 1), jnp.float32),
        ),
    )(x, router_W)
    eid = eid2[:, 0]
    wt = wt2[:, 0]

    perm = jnp.argsort(eid)
    counts = jnp.zeros((E,), jnp.int32).at[eid].add(1)
    offsets = jnp.concatenate(
        [jnp.zeros((1,), jnp.int32), jnp.cumsum(counts)[:-1]]
    )
    nblk = (counts + BT - 1) // BT
    cumblk = jnp.cumsum(nblk)
    sblk = (cumblk - nblk).astype(jnp.int32)
    total_blocks = cumblk[-1]
    jarr = jnp.arange(NB, dtype=jnp.int32)
    ej = jnp.searchsorted(cumblk, jarr, side="right").astype(jnp.int32)
    ej = jnp.where(jarr < total_blocks, ej, E - 1)
    within = jarr - (cumblk[ej] - nblk[ej])
    start = offsets[ej] + within * BT
    cnt = jnp.clip(counts[ej] - within * BT, 0, BT)
    cnt = jnp.where(jarr < total_blocks, cnt, 0)
    g = start[:, None] + jnp.arange(BT, dtype=jnp.int32)[None, :]
    validm = jnp.arange(BT, dtype=jnp.int32)[None, :] < cnt[:, None]
    tok = jnp.where(validm, perm[jnp.clip(g, 0, T - 1)], 0).astype(jnp.int32)
    tokf = tok.reshape(TP)
    validf = validm.reshape(TP)
    wblk = jnp.where(validf, wt[tokf], 0.0).astype(jnp.float32)
    pos = (
        jnp.zeros((T + 8,), jnp.int32)
        .at[jnp.where(validf, tokf, T)]
        .set(jnp.arange(TP, dtype=jnp.int32))[:T]
    )

    grid_spec = pltpu.PrefetchScalarGridSpec(
        num_scalar_prefetch=2,
        grid=(1,),
        in_specs=[
            pl.BlockSpec((NB, BT), lambda i, nb, sb: (0, 0)),
            pl.BlockSpec((NB, BT), lambda i, nb, sb: (0, 0)),
            pl.BlockSpec((T, D), lambda i, nb, sb: (0, 0)),
            pl.BlockSpec(memory_space=pl.ANY),
            pl.BlockSpec(memory_space=pl.ANY),
            pl.BlockSpec(memory_space=pl.ANY),
        ],
        out_specs=pl.BlockSpec(memory_space=pl.ANY),
        scratch_shapes=[
            pltpu.VMEM((2, DFF, D), jnp.float32),
            pltpu.VMEM((2, DFF, D), jnp.float32),
            pltpu.VMEM((2, D, DFF), jnp.float32),
            pltpu.VMEM((2, BT, D), jnp.float32),
            pltpu.SemaphoreType.DMA((2,)),
            pltpu.SemaphoreType.DMA((2,)),
        ],
    )
    y_sorted = pl.pallas_call(
        _moe_body,
        grid_spec=grid_spec,
        out_shape=jax.ShapeDtypeStruct((TP, D), jnp.float32),
        compiler_params=pltpu.CompilerParams(
            dimension_semantics=("arbitrary",),
            vmem_limit_bytes=120 * 1024 * 1024,
        ),
    )(
        nblk.astype(jnp.int32),
        sblk,
        tok,
        wblk.reshape(NB, BT),
        x,
        gate_W,
        up_W,
        down_W,
    )

    out = _combine_gather(y_sorted, pos)
    return out.reshape(b, s, d)


# speedup vs baseline: 1.0091x; 1.0091x over previous
import functools

import jax
import jax.numpy as jnp
from jax.experimental import pallas as pl
from jax.experimental.pallas import tpu as pltpu
from jax.experimental.pallas import tpu_sc as plsc

E = 64
D = 768
DFF = 2048
T = 2048
BT = 64
NB = T // BT + E
TP = NB * BT

_NC, _NS = 2, 16
_NW = _NC * _NS


def _routing_body(x_ref, rw_ref, eid_ref, wt_ref):
    x = x_ref[...]
    rw = rw_ref[...]
    logits = jax.lax.dot_general(
        x, rw, (((1,), (1,)), ((), ())),
        preferred_element_type=jnp.float32,
        precision=jax.lax.Precision.DEFAULT,
    )
    lmax = jnp.max(logits, axis=1, keepdims=True)
    sumexp = jnp.sum(jnp.exp(logits - lmax), axis=1, keepdims=True)
    iota = jax.lax.broadcasted_iota(jnp.int32, (T, E), 1)
    eid = jnp.min(jnp.where(logits == lmax, iota, E), axis=1, keepdims=True)
    eid_ref[...] = eid
    wt_ref[...] = 1.0 / sumexp


def _make_sc_row_gather(n_out, chunk):
    per_w = n_out // _NW
    nchunks = per_w // chunk
    mesh = plsc.VectorSubcoreMesh(
        core_axis_name="c",
        subcore_axis_name="s",
        num_cores=_NC,
        num_subcores=_NS,
    )

    @functools.partial(
        pl.kernel,
        mesh=mesh,
        out_type=jax.ShapeDtypeStruct((n_out, D), jnp.float32),
        scratch_types=[
            pltpu.VMEM((chunk,), jnp.int32),
            pltpu.VMEM((chunk, D), jnp.float32),
            pltpu.SemaphoreType.DMA,
        ],
    )
    def k(src_hbm, idx_hbm, out_hbm, idx_v, rows_v, sem):
        wid = jax.lax.axis_index("s") * _NC + jax.lax.axis_index("c")
        base = wid * per_w
        for c in range(nchunks):
            off = base + c * chunk
            pltpu.sync_copy(idx_hbm.at[pl.ds(off, chunk)], idx_v)
            pltpu.async_copy(src_hbm.at[idx_v], rows_v, sem).wait()
            pltpu.sync_copy(rows_v, out_hbm.at[pl.ds(off, chunk)])

    return k


_combine_gather = _make_sc_row_gather(T, BT)


def _y_copy(ybuf, y_hbm, sem, slot, row):
    return pltpu.make_async_copy(
        ybuf.at[slot], y_hbm.at[pl.ds(row * BT, BT), :], sem.at[slot]
    )


def _moe_body(
    nblk_ref,
    sblk_ref,
    tok_ref,
    wblk_ref,
    x_ref,
    g_hbm,
    u_hbm,
    d_hbm,
    y_hbm,
    x16,
    gbuf,
    ubuf,
    dbuf,
    ybuf,
    wsem,
    ysem,
):
    x16[...] = x_ref[...].astype(jnp.bfloat16)

    def w_copies(ei, sl):
        cs = []
        for p in range(4):
            ff = pl.ds(p * (DFF // 4), DFF // 4)
            dd = pl.ds(p * (D // 4), D // 4)
            cs.append(
                pltpu.make_async_copy(
                    g_hbm.at[ei, ff, :], gbuf.at[sl, ff, :], wsem.at[sl]
                )
            )
            cs.append(
                pltpu.make_async_copy(
                    u_hbm.at[ei, ff, :], ubuf.at[sl, ff, :], wsem.at[sl]
                )
            )
            cs.append(
                pltpu.make_async_copy(
                    d_hbm.at[ei, dd, :], dbuf.at[sl, dd, :], wsem.at[sl]
                )
            )
        return cs

    for c in w_copies(0, 0):
        c.start()

    def expert_step(e, carry):
        slot = jax.lax.rem(e, 2)

        @pl.when(e + 1 < E)
        def _():
            for c in w_copies(e + 1, jax.lax.rem(e + 1, 2)):
                c.start()

        for c in w_copies(e, slot):
            c.wait()

        gw = gbuf[slot].astype(jnp.bfloat16)
        uw = ubuf[slot].astype(jnp.bfloat16)
        dw = dbuf[slot].astype(jnp.bfloat16)
        s0 = sblk_ref[e]
        n = nblk_ref[e]

        def sub_block(k, kc):
            row = s0 + k
            yslot = jax.lax.rem(k, 2)
            idx = tok_ref[pl.ds(row, 1), :][0]
            w = wblk_ref[pl.ds(row, 1), :][0]
            iota_bt = jax.lax.broadcasted_iota(jnp.int32, (BT, T), 1)
            gat = (iota_bt == idx[:, None]).astype(jnp.bfloat16)
            xb = jax.lax.dot_general(
                gat, x16[...], (((1,), (0,)), ((), ())),
                preferred_element_type=jnp.float32,
                precision=jax.lax.Precision.DEFAULT,
            )
            xb = xb.astype(jnp.bfloat16)
            hg = jax.lax.dot_general(
                xb, gw, (((1,), (1,)), ((), ())),
                preferred_element_type=jnp.float32,
                precision=jax.lax.Precision.DEFAULT,
            )
            hu = jax.lax.dot_general(
                xb, uw, (((1,), (1,)), ((), ())),
                preferred_element_type=jnp.float32,
                precision=jax.lax.Precision.DEFAULT,
            )
            h = (hg * jax.nn.sigmoid(hg) * hu).astype(jnp.bfloat16)
            y = jax.lax.dot_general(
                h, dw, (((1,), (1,)), ((), ())),
                preferred_element_type=jnp.float32,
                precision=jax.lax.Precision.DEFAULT,
            )

            @pl.when(k >= 2)
            def _():
                _y_copy(ybuf, y_hbm, ysem, yslot, row - 2).wait()

            ybuf[pl.ds(yslot, 1), :, :] = (y * w[:, None])[None]
            _y_copy(ybuf, y_hbm, ysem, yslot, row).start()
            return kc

        jax.lax.fori_loop(0, n, sub_block, 0)

        @pl.when(n >= 2)
        def _():
            _y_copy(ybuf, y_hbm, ysem, jax.lax.rem(n - 2, 2), s0 + n - 2).wait()

        @pl.when(n >= 1)
        def _():
            _y_copy(ybuf, y_hbm, ysem, jax.lax.rem(n - 1, 2), s0 + n - 1).wait()

        return carry

    jax.lax.fori_loop(0, E, expert_step, 0)


@jax.jit
def kernel(hidden_states, router_W, gate_W, up_W, down_W):
    b, s, d = hidden_states.shape
    x = hidden_states.reshape(-1, d).astype(jnp.float32)

    eid2, wt2 = pl.pallas_call(
        _routing_body,
        out_shape=(
            jax.ShapeDtypeStruct((T, 1), jnp.int32),
            jax.ShapeDtypeStruct((T, 1), jnp.float32),
        ),
    )(x, router_W)
    eid = eid2[:, 0]
    wt = wt2[:, 0]

    perm = jnp.argsort(eid)
    counts = jnp.zeros((E,), jnp.int32).at[eid].add(1)
    offsets = jnp.concatenate(
        [jnp.zeros((1,), jnp.int32), jnp.cumsum(counts)[:-1]]
    )
    nblk = (counts + BT - 1) // BT
    cumblk = jnp.cumsum(nblk)
    sblk = (cumblk - nblk).astype(jnp.int32)
    total_blocks = cumblk[-1]
    jarr = jnp.arange(NB, dtype=jnp.int32)
    ej = jnp.searchsorted(cumblk, jarr, side="right").astype(jnp.int32)
    ej = jnp.where(jarr < total_blocks, ej, E - 1)
    within = jarr - (cumblk[ej] - nblk[ej])
    start = offsets[ej] + within * BT
    cnt = jnp.clip(counts[ej] - within * BT, 0, BT)
    cnt = jnp.where(jarr < total_blocks, cnt, 0)
    g = start[:, None] + jnp.arange(BT, dtype=jnp.int32)[None, :]
    validm = jnp.arange(BT, dtype=jnp.int32)[None, :] < cnt[:, None]
    tok = jnp.where(validm, perm[jnp.clip(g, 0, T - 1)], 0).astype(jnp.int32)
    tokf = tok.reshape(TP)
    validf = validm.reshape(TP)
    wblk = jnp.where(validf, wt[tokf], 0.0).astype(jnp.float32)
    pos = (
        jnp.zeros((T + 8,), jnp.int32)
        .at[jnp.where(validf, tokf, T)]
        .set(jnp.arange(TP, dtype=jnp.int32))[:T]
    )

    grid_spec = pltpu.PrefetchScalarGridSpec(
        num_scalar_prefetch=2,
        grid=(1,),
        in_specs=[
            pl.BlockSpec((NB, BT), lambda i, nb, sb: (0, 0)),
            pl.BlockSpec((NB, BT), lambda i, nb, sb: (0, 0)),
            pl.BlockSpec((T, D), lambda i, nb, sb: (0, 0)),
            pl.BlockSpec(memory_space=pl.ANY),
            pl.BlockSpec(memory_space=pl.ANY),
            pl.BlockSpec(memory_space=pl.ANY),
        ],
        out_specs=pl.BlockSpec(memory_space=pl.ANY),
        scratch_shapes=[
            pltpu.VMEM((T, D), jnp.bfloat16),
            pltpu.VMEM((2, DFF, D), jnp.float32),
            pltpu.VMEM((2, DFF, D), jnp.float32),
            pltpu.VMEM((2, D, DFF), jnp.float32),
            pltpu.VMEM((2, BT, D), jnp.float32),
            pltpu.SemaphoreType.DMA((2,)),
            pltpu.SemaphoreType.DMA((2,)),
        ],
    )
    y_sorted = pl.pallas_call(
        _moe_body,
        grid_spec=grid_spec,
        out_shape=jax.ShapeDtypeStruct((TP, D), jnp.float32),
        compiler_params=pltpu.CompilerParams(
            dimension_semantics=("arbitrary",),
            vmem_limit_bytes=120 * 1024 * 1024,
        ),
    )(
        nblk.astype(jnp.int32),
        sblk,
        tok,
        wblk.reshape(NB, BT),
        x,
        gate_W,
        up_W,
        down_W,
    )

    out = _combine_gather(y_sorted, pos)
    return out.reshape(b, s, d)


# speedup vs baseline: 1.0399x; 1.0305x over previous
import functools

import jax
import jax.numpy as jnp
from jax.experimental import pallas as pl
from jax.experimental.pallas import tpu as pltpu
from jax.experimental.pallas import tpu_sc as plsc

E = 64
D = 768
DFF = 2048
T = 2048
BT = 64
NB = T // BT + E
TP = NB * BT

_NC, _NS = 2, 16
_NW = _NC * _NS


def _routing_body(x_ref, rw_ref, eid_ref, wt_ref):
    x = x_ref[...]
    rw = rw_ref[...]
    logits = jax.lax.dot_general(
        x, rw, (((1,), (1,)), ((), ())),
        preferred_element_type=jnp.float32,
        precision=jax.lax.Precision.DEFAULT,
    )
    lmax = jnp.max(logits, axis=1, keepdims=True)
    sumexp = jnp.sum(jnp.exp(logits - lmax), axis=1, keepdims=True)
    iota = jax.lax.broadcasted_iota(jnp.int32, (T, E), 1)
    eid = jnp.min(jnp.where(logits == lmax, iota, E), axis=1, keepdims=True)
    eid_ref[...] = eid
    wt_ref[...] = 1.0 / sumexp


def _make_sc_row_gather(n_out, chunk):
    per_w = n_out // _NW
    nchunks = per_w // chunk
    mesh = plsc.VectorSubcoreMesh(
        core_axis_name="c",
        subcore_axis_name="s",
        num_cores=_NC,
        num_subcores=_NS,
    )

    @functools.partial(
        pl.kernel,
        mesh=mesh,
        out_type=jax.ShapeDtypeStruct((n_out, D), jnp.float32),
        scratch_types=[
            pltpu.VMEM((chunk,), jnp.int32),
            pltpu.VMEM((chunk, D), jnp.float32),
            pltpu.SemaphoreType.DMA,
        ],
    )
    def k(src_hbm, idx_hbm, out_hbm, idx_v, rows_v, sem):
        wid = jax.lax.axis_index("s") * _NC + jax.lax.axis_index("c")
        base = wid * per_w
        for c in range(nchunks):
            off = base + c * chunk
            pltpu.sync_copy(idx_hbm.at[pl.ds(off, chunk)], idx_v)
            pltpu.async_copy(src_hbm.at[idx_v], rows_v, sem).wait()
            pltpu.sync_copy(rows_v, out_hbm.at[pl.ds(off, chunk)])

    return k


_combine_gather = _make_sc_row_gather(T, BT)


def _y_copy(ybuf, y_hbm, sem, slot, row):
    return pltpu.make_async_copy(
        ybuf.at[slot], y_hbm.at[pl.ds(row * BT, BT), :], sem.at[slot]
    )


def _moe_body(
    nblk_ref,
    sblk_ref,
    tok_ref,
    wblk_ref,
    x_ref,
    g_hbm,
    u_hbm,
    d_hbm,
    y_hbm,
    gbuf,
    ubuf,
    dbuf,
    ybuf,
    wsem,
    ysem,
    rowring,
):
    def w_copies(ei, sl):
        cs = []
        for p in range(4):
            ff = pl.ds(p * (DFF // 4), DFF // 4)
            dd = pl.ds(p * (D // 4), D // 4)
            cs.append(
                pltpu.make_async_copy(
                    g_hbm.at[ei, ff, :], gbuf.at[sl, ff, :], wsem.at[sl]
                )
            )
            cs.append(
                pltpu.make_async_copy(
                    u_hbm.at[ei, ff, :], ubuf.at[sl, ff, :], wsem.at[sl]
                )
            )
            cs.append(
                pltpu.make_async_copy(
                    d_hbm.at[ei, dd, :], dbuf.at[sl, dd, :], wsem.at[sl]
                )
            )
        return cs

    for c in w_copies(0, 0):
        c.start()

    def expert_step(e, q0):
        slot = jax.lax.rem(e, 2)

        @pl.when(e + 1 < E)
        def _():
            for c in w_copies(e + 1, jax.lax.rem(e + 1, 2)):
                c.start()

        for c in w_copies(e, slot):
            c.wait()

        gw = gbuf[slot]
        uw = ubuf[slot]
        dw = dbuf[slot]
        s0 = sblk_ref[e]
        n = nblk_ref[e]

        def sub_block(k, q):
            row = s0 + k
            yslot = jax.lax.rem(q, 2)
            idx = tok_ref[pl.ds(row, 1), :][0]
            w = wblk_ref[pl.ds(row, 1), :][0]
            iota_bt = jax.lax.broadcasted_iota(jnp.int32, (BT, T), 1)
            gat = (iota_bt == idx[:, None]).astype(jnp.float32)
            xb = jax.lax.dot_general(
                gat, x_ref[...], (((1,), (0,)), ((), ())),
                preferred_element_type=jnp.float32,
                precision=jax.lax.Precision.DEFAULT,
            )
            hg = jax.lax.dot_general(
                xb, gw, (((1,), (1,)), ((), ())),
                preferred_element_type=jnp.float32,
                precision=jax.lax.Precision.DEFAULT,
            )
            hu = jax.lax.dot_general(
                xb, uw, (((1,), (1,)), ((), ())),
                preferred_element_type=jnp.float32,
                precision=jax.lax.Precision.DEFAULT,
            )
            h = hg * jax.nn.sigmoid(hg) * hu
            y = jax.lax.dot_general(
                h, dw, (((1,), (1,)), ((), ())),
                preferred_element_type=jnp.float32,
                precision=jax.lax.Precision.DEFAULT,
            )

            @pl.when(q >= 2)
            def _():
                _y_copy(ybuf, y_hbm, ysem, yslot, rowring[yslot]).wait()

            ybuf[pl.ds(yslot, 1), :, :] = (y * w[:, None])[None]
            _y_copy(ybuf, y_hbm, ysem, yslot, row).start()
            rowring[yslot] = row
            return q + 1

        return jax.lax.fori_loop(0, n, sub_block, q0)

    qf = jax.lax.fori_loop(0, E, expert_step, 0)

    @pl.when(qf >= 2)
    def _():
        sl = jax.lax.rem(qf, 2)
        _y_copy(ybuf, y_hbm, ysem, sl, rowring[sl]).wait()

    @pl.when(qf >= 1)
    def _():
        sl = jax.lax.rem(qf - 1, 2)
        _y_copy(ybuf, y_hbm, ysem, sl, rowring[sl]).wait()


@jax.jit
def kernel(hidden_states, router_W, gate_W, up_W, down_W):
    b, s, d = hidden_states.shape
    x = hidden_states.reshape(-1, d).astype(jnp.float32)

    eid2, wt2 = pl.pallas_call(
        _routing_body,
        out_shape=(
            jax.ShapeDtypeStruct((T, 1), jnp.int32),
            jax.ShapeDtypeStruct((T, 1), jnp.float32),
        ),
    )(x, router_W)
    eid = eid2[:, 0]
    wt = wt2[:, 0]

    perm = jnp.argsort(eid)
    counts = jnp.zeros((E,), jnp.int32).at[eid].add(1)
    offsets = jnp.concatenate(
        [jnp.zeros((1,), jnp.int32), jnp.cumsum(counts)[:-1]]
    )
    nblk = (counts + BT - 1) // BT
    cumblk = jnp.cumsum(nblk)
    sblk = (cumblk - nblk).astype(jnp.int32)
    total_blocks = cumblk[-1]
    jarr = jnp.arange(NB, dtype=jnp.int32)
    ej = jnp.searchsorted(cumblk, jarr, side="right").astype(jnp.int32)
    ej = jnp.where(jarr < total_blocks, ej, E - 1)
    within = jarr - (cumblk[ej] - nblk[ej])
    start = offsets[ej] + within * BT
    cnt = jnp.clip(counts[ej] - within * BT, 0, BT)
    cnt = jnp.where(jarr < total_blocks, cnt, 0)
    g = start[:, None] + jnp.arange(BT, dtype=jnp.int32)[None, :]
    validm = jnp.arange(BT, dtype=jnp.int32)[None, :] < cnt[:, None]
    tok = jnp.where(validm, perm[jnp.clip(g, 0, T - 1)], 0).astype(jnp.int32)
    tokf = tok.reshape(TP)
    validf = validm.reshape(TP)
    wblk = jnp.where(validf, wt[tokf], 0.0).astype(jnp.float32)
    pos = (
        jnp.zeros((T + 8,), jnp.int32)
        .at[jnp.where(validf, tokf, T)]
        .set(jnp.arange(TP, dtype=jnp.int32))[:T]
    )

    grid_spec = pltpu.PrefetchScalarGridSpec(
        num_scalar_prefetch=2,
        grid=(1,),
        in_specs=[
            pl.BlockSpec((NB, BT), lambda i, nb, sb: (0, 0)),
            pl.BlockSpec((NB, BT), lambda i, nb, sb: (0, 0)),
            pl.BlockSpec((T, D), lambda i, nb, sb: (0, 0)),
            pl.BlockSpec(memory_space=pl.ANY),
            pl.BlockSpec(memory_space=pl.ANY),
            pl.BlockSpec(memory_space=pl.ANY),
        ],
        out_specs=pl.BlockSpec(memory_space=pl.ANY),
        scratch_shapes=[
            pltpu.VMEM((2, DFF, D), jnp.float32),
            pltpu.VMEM((2, DFF, D), jnp.float32),
            pltpu.VMEM((2, D, DFF), jnp.float32),
            pltpu.VMEM((2, BT, D), jnp.float32),
            pltpu.SemaphoreType.DMA((2,)),
            pltpu.SemaphoreType.DMA((2,)),
            pltpu.SMEM((2,), jnp.int32),
        ],
    )
    y_sorted = pl.pallas_call(
        _moe_body,
        grid_spec=grid_spec,
        out_shape=jax.ShapeDtypeStruct((TP, D), jnp.float32),
        compiler_params=pltpu.CompilerParams(
            dimension_semantics=("arbitrary",),
            vmem_limit_bytes=120 * 1024 * 1024,
        ),
    )(
        nblk.astype(jnp.int32),
        sblk,
        tok,
        wblk.reshape(NB, BT),
        x,
        gate_W,
        up_W,
        down_W,
    )

    out = _combine_gather(y_sorted, pos)
    return out.reshape(b, s, d)


# speedup vs baseline: 1.0435x; 1.0034x over previous
import functools

import jax
import jax.numpy as jnp
from jax.experimental import pallas as pl
from jax.experimental.pallas import tpu as pltpu
from jax.experimental.pallas import tpu_sc as plsc

E = 64
D = 768
DFF = 2048
T = 2048
BT = 64
NB = T // BT + E
TP = NB * BT

_NC, _NS = 2, 16
_NW = _NC * _NS


def _routing_body(x_ref, rw_ref, eid_ref, wt_ref):
    x = x_ref[...]
    rw = rw_ref[...]
    logits = jax.lax.dot_general(
        x, rw, (((1,), (1,)), ((), ())),
        preferred_element_type=jnp.float32,
        precision=jax.lax.Precision.DEFAULT,
    )
    lmax = jnp.max(logits, axis=1, keepdims=True)
    sumexp = jnp.sum(jnp.exp(logits - lmax), axis=1, keepdims=True)
    iota = jax.lax.broadcasted_iota(jnp.int32, (T, E), 1)
    eid = jnp.min(jnp.where(logits == lmax, iota, E), axis=1, keepdims=True)
    eid_ref[...] = eid
    wt_ref[...] = 1.0 / sumexp


def _make_sc_row_gather(n_out, chunk):
    per_w = n_out // _NW
    nchunks = per_w // chunk
    mesh = plsc.VectorSubcoreMesh(
        core_axis_name="c",
        subcore_axis_name="s",
        num_cores=_NC,
        num_subcores=_NS,
    )

    @functools.partial(
        pl.kernel,
        mesh=mesh,
        out_type=jax.ShapeDtypeStruct((n_out, D), jnp.float32),
        scratch_types=[
            pltpu.VMEM((chunk,), jnp.int32),
            pltpu.VMEM((chunk, D), jnp.float32),
            pltpu.SemaphoreType.DMA,
        ],
    )
    def k(src_hbm, idx_hbm, out_hbm, idx_v, rows_v, sem):
        wid = jax.lax.axis_index("s") * _NC + jax.lax.axis_index("c")
        base = wid * per_w
        for c in range(nchunks):
            off = base + c * chunk
            pltpu.sync_copy(idx_hbm.at[pl.ds(off, chunk)], idx_v)
            pltpu.async_copy(src_hbm.at[idx_v], rows_v, sem).wait()
            pltpu.sync_copy(rows_v, out_hbm.at[pl.ds(off, chunk)])

    return k


_combine_gather = _make_sc_row_gather(T, BT)


def _y_copy(ybuf, y_hbm, sem, slot, row):
    return pltpu.make_async_copy(
        ybuf.at[slot], y_hbm.at[pl.ds(row * BT, BT), :], sem.at[slot]
    )


def _moe_body(
    nblk_ref,
    sblk_ref,
    toks_ref,
    wblk_ref,
    x_ref,
    g_hbm,
    u_hbm,
    d_hbm,
    y_hbm,
    gbuf,
    ubuf,
    dbuf,
    xbbuf,
    ybuf,
    wsem,
    ysem,
    rowring,
):
    def w_copies(ei, sl):
        cs = []
        for p in range(4):
            ff = pl.ds(p * (DFF // 4), DFF // 4)
            dd = pl.ds(p * (D // 4), D // 4)
            cs.append(
                pltpu.make_async_copy(
                    g_hbm.at[ei, ff, :], gbuf.at[sl, ff, :], wsem.at[sl]
                )
            )
            cs.append(
                pltpu.make_async_copy(
                    u_hbm.at[ei, ff, :], ubuf.at[sl, ff, :], wsem.at[sl]
                )
            )
            cs.append(
                pltpu.make_async_copy(
                    d_hbm.at[ei, dd, :], dbuf.at[sl, dd, :], wsem.at[sl]
                )
            )
        return cs

    for c in w_copies(0, 0):
        c.start()

    def expert_step(e, q0):
        slot = jax.lax.rem(e, 2)

        @pl.when(e + 1 < E)
        def _():
            for c in w_copies(e + 1, jax.lax.rem(e + 1, 2)):
                c.start()

        for c in w_copies(e, slot):
            c.wait()

        gw = gbuf[slot]
        uw = ubuf[slot]
        dw = dbuf[slot]
        s0 = sblk_ref[e]
        n = nblk_ref[e]

        def sub_block(k, q):
            row = s0 + k
            yslot = jax.lax.rem(q, 2)
            w = wblk_ref[pl.ds(row, 1), :][0]

            def grow(i, c):
                r = toks_ref[row * BT + i]
                xbbuf[pl.ds(i, 1), :] = x_ref[pl.ds(r, 1), :]
                return c

            jax.lax.fori_loop(0, BT, grow, 0)
            xb = xbbuf[...]
            hg = jax.lax.dot_general(
                xb, gw, (((1,), (1,)), ((), ())),
                preferred_element_type=jnp.float32,
                precision=jax.lax.Precision.DEFAULT,
            )
            hu = jax.lax.dot_general(
                xb, uw, (((1,), (1,)), ((), ())),
                preferred_element_type=jnp.float32,
                precision=jax.lax.Precision.DEFAULT,
            )
            h = hg * jax.nn.sigmoid(hg) * hu
            y = jax.lax.dot_general(
                h, dw, (((1,), (1,)), ((), ())),
                preferred_element_type=jnp.float32,
                precision=jax.lax.Precision.DEFAULT,
            )

            @pl.when(q >= 2)
            def _():
                _y_copy(ybuf, y_hbm, ysem, yslot, rowring[yslot]).wait()

            ybuf[pl.ds(yslot, 1), :, :] = (y * w[:, None])[None]
            _y_copy(ybuf, y_hbm, ysem, yslot, row).start()
            rowring[yslot] = row
            return q + 1

        return jax.lax.fori_loop(0, n, sub_block, q0)

    qf = jax.lax.fori_loop(0, E, expert_step, 0)

    @pl.when(qf >= 2)
    def _():
        sl = jax.lax.rem(qf, 2)
        _y_copy(ybuf, y_hbm, ysem, sl, rowring[sl]).wait()

    @pl.when(qf >= 1)
    def _():
        sl = jax.lax.rem(qf - 1, 2)
        _y_copy(ybuf, y_hbm, ysem, sl, rowring[sl]).wait()


@jax.jit
def kernel(hidden_states, router_W, gate_W, up_W, down_W):
    b, s, d = hidden_states.shape
    x = hidden_states.reshape(-1, d).astype(jnp.float32)

    eid2, wt2 = pl.pallas_call(
        _routing_body,
        out_shape=(
            jax.ShapeDtypeStruct((T, 1), jnp.int32),
            jax.ShapeDtypeStruct((T, 1), jnp.float32),
        ),
    )(x, router_W)
    eid = eid2[:, 0]
    wt = wt2[:, 0]

    perm = jnp.argsort(eid)
    counts = jnp.zeros((E,), jnp.int32).at[eid].add(1)
    offsets = jnp.concatenate(
        [jnp.zeros((1,), jnp.int32), jnp.cumsum(counts)[:-1]]
    )
    nblk = (counts + BT - 1) // BT
    cumblk = jnp.cumsum(nblk)
    sblk = (cumblk - nblk).astype(jnp.int32)
    total_blocks = cumblk[-1]
    jarr = jnp.arange(NB, dtype=jnp.int32)
    ej = jnp.searchsorted(cumblk, jarr, side="right").astype(jnp.int32)
    ej = jnp.where(jarr < total_blocks, ej, E - 1)
    within = jarr - (cumblk[ej] - nblk[ej])
    start = offsets[ej] + within * BT
    cnt = jnp.clip(counts[ej] - within * BT, 0, BT)
    cnt = jnp.where(jarr < total_blocks, cnt, 0)
    g = start[:, None] + jnp.arange(BT, dtype=jnp.int32)[None, :]
    validm = jnp.arange(BT, dtype=jnp.int32)[None, :] < cnt[:, None]
    tok = jnp.where(validm, perm[jnp.clip(g, 0, T - 1)], 0).astype(jnp.int32)
    tokf = tok.reshape(TP)
    validf = validm.reshape(TP)
    wblk = jnp.where(validf, wt[tokf], 0.0).astype(jnp.float32)
    pos = (
        jnp.zeros((T + 8,), jnp.int32)
        .at[jnp.where(validf, tokf, T)]
        .set(jnp.arange(TP, dtype=jnp.int32))[:T]
    )

    grid_spec = pltpu.PrefetchScalarGridSpec(
        num_scalar_prefetch=3,
        grid=(1,),
        in_specs=[
            pl.BlockSpec((NB, BT), lambda i, nb, sb, tk: (0, 0)),
            pl.BlockSpec((T, D), lambda i, nb, sb, tk: (0, 0)),
            pl.BlockSpec(memory_space=pl.ANY),
            pl.BlockSpec(memory_space=pl.ANY),
            pl.BlockSpec(memory_space=pl.ANY),
        ],
        out_specs=pl.BlockSpec(memory_space=pl.ANY),
        scratch_shapes=[
            pltpu.VMEM((2, DFF, D), jnp.float32),
            pltpu.VMEM((2, DFF, D), jnp.float32),
            pltpu.VMEM((2, D, DFF), jnp.float32),
            pltpu.VMEM((BT, D), jnp.float32),
            pltpu.VMEM((2, BT, D), jnp.float32),
            pltpu.SemaphoreType.DMA((2,)),
            pltpu.SemaphoreType.DMA((2,)),
            pltpu.SMEM((2,), jnp.int32),
        ],
    )
    y_sorted = pl.pallas_call(
        _moe_body,
        grid_spec=grid_spec,
        out_shape=jax.ShapeDtypeStruct((TP, D), jnp.float32),
        compiler_params=pltpu.CompilerParams(
            dimension_semantics=("arbitrary",),
            vmem_limit_bytes=120 * 1024 * 1024,
        ),
    )(
        nblk.astype(jnp.int32),
        sblk,
        tokf,
        wblk.reshape(NB, BT),
        x,
        gate_W,
        up_W,
        down_W,
    )

    out = _combine_gather(y_sorted, pos)
    return out.reshape(b, s, d)
